# baseline (speedup 1.0000x reference)
import functools

import jax
import jax.numpy as jnp
from jax import lax
from jax.experimental import pallas as pl
from jax.experimental.pallas import tpu as pltpu
from jax.experimental.pallas import tpu_sc as plsc

B = 4096
S = 200
EMB = 64
N = B * S
NC = 2
NS = 16
NW = NC * NS
PER_W = N // NW
CHUNK = 800
NCHUNK = PER_W // CHUNK

_mesh = plsc.VectorSubcoreMesh(core_axis_name="c", subcore_axis_name="s")


@functools.partial(
    pl.kernel,
    out_type=jax.ShapeDtypeStruct((N, EMB), jnp.float32),
    mesh=_mesh,
    scratch_types=[
        pltpu.VMEM((CHUNK,), jnp.int32),
        pltpu.VMEM((CHUNK, EMB), jnp.float32),
        pltpu.SemaphoreType.DMA,
    ],
    compiler_params=pltpu.CompilerParams(use_tc_tiling_on_sc=False),
)
def _gather_kernel(idx_hbm, table_hbm, out_hbm, idx_v, rows_v, sem):
    wid = lax.axis_index("s") * NC + lax.axis_index("c")
    base = wid * PER_W

    def step(i, carry):
        off = pl.multiple_of(base + i * CHUNK, 8)
        pltpu.sync_copy(idx_hbm.at[pl.ds(off, CHUNK)], idx_v)
        pltpu.async_copy(table_hbm.at[idx_v], rows_v, sem).wait()
        pltpu.sync_copy(rows_v, out_hbm.at[pl.ds(off, CHUNK)])
        return carry

    lax.fori_loop(0, NCHUNK, step, 0)


def kernel(x, table):
    xf = x.reshape(-1).astype(jnp.int32)
    out = _gather_kernel(xf, table)
    return out.reshape(B, S, EMB)


# speedup vs baseline: 1.0208x; 1.0208x over previous
import functools

import jax
import jax.numpy as jnp
from jax import lax
from jax.experimental import pallas as pl
from jax.experimental.pallas import tpu as pltpu
from jax.experimental.pallas import tpu_sc as plsc

B = 4096
S = 200
EMB = 64
N = B * S
NC = 2
NS = 16
NW = NC * NS
PER_W = N // NW
CHUNK = 800
NCHUNK = PER_W // CHUNK
NPAIR = NCHUNK // 2

_mesh = plsc.VectorSubcoreMesh(core_axis_name="c", subcore_axis_name="s")


@functools.partial(
    pl.kernel,
    out_type=jax.ShapeDtypeStruct((N, EMB), jnp.float32),
    mesh=_mesh,
    scratch_types=[
        pltpu.VMEM((CHUNK,), jnp.int32),
        pltpu.VMEM((CHUNK,), jnp.int32),
        pltpu.VMEM((CHUNK, EMB), jnp.float32),
        pltpu.VMEM((CHUNK, EMB), jnp.float32),
        pltpu.SemaphoreType.DMA,
        pltpu.SemaphoreType.DMA,
        pltpu.SemaphoreType.DMA,
        pltpu.SemaphoreType.DMA,
        pltpu.SemaphoreType.DMA,
        pltpu.SemaphoreType.DMA,
    ],
    compiler_params=pltpu.CompilerParams(use_tc_tiling_on_sc=False),
)
def _gather_kernel(idx_hbm, table_hbm, out_hbm,
                   idx0, idx1, rows0, rows1,
                   isem0, isem1, gsem0, gsem1, ssem0, ssem1):
    wid = lax.axis_index("s") * NC + lax.axis_index("c")
    base = wid * PER_W
    idx_v = (idx0, idx1)
    rows_v = (rows0, rows1)
    isem = (isem0, isem1)
    gsem = (gsem0, gsem1)
    ssem = (ssem0, ssem1)

    def off(i):
        return pl.multiple_of(base + i * CHUNK, 8)

    pltpu.async_copy(idx_hbm.at[pl.ds(off(0), CHUNK)], idx0, isem0)
    pltpu.async_copy(idx_hbm.at[pl.ds(off(1), CHUNK)], idx1, isem1)
    pltpu.make_async_copy(idx_hbm.at[pl.ds(off(0), CHUNK)], idx0, isem0).wait()
    pltpu.async_copy(table_hbm.at[idx0], rows0, gsem0)
    pltpu.make_async_copy(idx_hbm.at[pl.ds(off(1), CHUNK)], idx1, isem1).wait()
    pltpu.async_copy(table_hbm.at[idx1], rows1, gsem1)

    def outer(t, carry):
        for b in range(2):
            i = 2 * t + b
            pltpu.make_async_copy(table_hbm.at[idx_v[b]], rows_v[b],
                                  gsem[b]).wait()
            pltpu.async_copy(rows_v[b], out_hbm.at[pl.ds(off(i), CHUNK)],
                             ssem[b])

            @pl.when(t < NPAIR - 1)
            def _(b=b, i=i):
                pltpu.async_copy(idx_hbm.at[pl.ds(off(i + 2), CHUNK)],
                                 idx_v[b], isem[b])

        @pl.when(t < NPAIR - 1)
        def _():
            for b in range(2):
                i2 = 2 * t + b + 2
                pltpu.make_async_copy(
                    rows_v[b], out_hbm.at[pl.ds(off(i2 - 2), CHUNK)],
                    ssem[b]).wait()
                pltpu.make_async_copy(
                    idx_hbm.at[pl.ds(off(i2), CHUNK)], idx_v[b],
                    isem[b]).wait()
                pltpu.async_copy(table_hbm.at[idx_v[b]], rows_v[b], gsem[b])

        return carry

    lax.fori_loop(0, NPAIR, outer, 0)

    for b in range(2):
        i = NCHUNK - 2 + b
        pltpu.make_async_copy(rows_v[b], out_hbm.at[pl.ds(off(i), CHUNK)],
                              ssem[b]).wait()


def kernel(x, table):
    xf = x.reshape(-1).astype(jnp.int32)
    out = _gather_kernel(xf, table)
    return out.reshape(B, S, EMB)


# speedup vs baseline: 1.0283x; 1.0074x over previous
import functools

import jax
import jax.numpy as jnp
from jax import lax
from jax.experimental import pallas as pl
from jax.experimental.pallas import tpu as pltpu
from jax.experimental.pallas import tpu_sc as plsc

B = 4096
S = 200
EMB = 64
NC = 2
NS = 16
NW = NC * NS
ROWS_W = B // NW
NBUF = 4
NOUT = ROWS_W // NBUF

_mesh = plsc.VectorSubcoreMesh(core_axis_name="c", subcore_axis_name="s")


@functools.partial(
    pl.kernel,
    out_type=jax.ShapeDtypeStruct((B, S, EMB), jnp.float32),
    mesh=_mesh,
    scratch_types=(
        [pltpu.VMEM((S,), jnp.int32) for _ in range(NBUF)]
        + [pltpu.VMEM((S, EMB), jnp.float32) for _ in range(NBUF)]
        + [pltpu.SemaphoreType.DMA for _ in range(3 * NBUF)]
    ),
    compiler_params=pltpu.CompilerParams(use_tc_tiling_on_sc=False),
)
def _gather_kernel(idx_hbm, table_hbm, out_hbm, *refs):
    idx_v = refs[0:NBUF]
    rows_v = refs[NBUF:2 * NBUF]
    isem = refs[2 * NBUF:3 * NBUF]
    gsem = refs[3 * NBUF:4 * NBUF]
    ssem = refs[4 * NBUF:5 * NBUF]

    wid = lax.axis_index("s") * NC + lax.axis_index("c")
    r0 = wid * ROWS_W

    for b in range(NBUF):
        pltpu.async_copy(idx_hbm.at[r0 + b], idx_v[b], isem[b])
    for b in range(NBUF):
        pltpu.make_async_copy(idx_hbm.at[r0 + b], idx_v[b], isem[b]).wait()
        pltpu.async_copy(table_hbm.at[idx_v[b]], rows_v[b], gsem[b])

    def outer(t, carry):
        for b in range(NBUF):
            r = r0 + NBUF * t + b
            pltpu.make_async_copy(table_hbm.at[idx_v[b]], rows_v[b],
                                  gsem[b]).wait()
            pltpu.async_copy(rows_v[b], out_hbm.at[r], ssem[b])

            @pl.when(t < NOUT - 1)
            def _(b=b, r=r):
                pltpu.async_copy(idx_hbm.at[r + NBUF], idx_v[b], isem[b])

        @pl.when(t < NOUT - 1)
        def _():
            for b in range(NBUF):
                r = r0 + NBUF * t + b
                pltpu.make_async_copy(rows_v[b], out_hbm.at[r],
                                      ssem[b]).wait()
                pltpu.make_async_copy(idx_hbm.at[r + NBUF], idx_v[b],
                                      isem[b]).wait()
                pltpu.async_copy(table_hbm.at[idx_v[b]], rows_v[b], gsem[b])

        return carry

    lax.fori_loop(0, NOUT, outer, 0)

    for b in range(NBUF):
        r = r0 + ROWS_W - NBUF + b
        pltpu.make_async_copy(rows_v[b], out_hbm.at[r], ssem[b]).wait()


def kernel(x, table):
    return _gather_kernel(x.astype(jnp.int32), table)
